# baseline (speedup 1.0000x reference)
import functools

import jax
import jax.numpy as jnp
from jax import lax
from jax.experimental import pallas as pl
from jax.experimental.pallas import tpu as pltpu
from jax.experimental.pallas import tpu_sc as plsc

_CUTOFF = 14.0
_EPS_RF = 78.4
_MRF = 4
_NRF = 6
_KRF = (_EPS_RF - 1) / (1 + 2 * _EPS_RF) * (1 / _CUTOFF ** 3)
_ARFM = 3 * _CUTOFF ** (-(_MRF + 1)) / (_MRF * (_NRF - _MRF)) * ((2 * _EPS_RF + _NRF - 1) / (1 + 2 * _EPS_RF))
_ARFN = 3 * _CUTOFF ** (-(_NRF + 1)) / (_NRF * (_MRF - _NRF)) * ((2 * _EPS_RF + _MRF - 1) / (1 + 2 * _EPS_RF))
_CRF = 3 * _EPS_RF / (1 + 2 * _EPS_RF) * (1 / _CUTOFF) + _ARFM * _CUTOFF ** _MRF + _ARFN * _CUTOFF ** _NRF
_K_EPS = 1389.35457644382

_NC = 2
_NS = 16
_NW = _NC * _NS
_L = 16

_N_EDGES = 3200000
_P = 2
_EP = _N_EDGES // _P
_PER_TILE = _EP // _NW
_C = 2000
_NCHUNK = _PER_TILE // _C
_FW = 16
_NE = 14


def _sc_body(f_hbm, e_hbms, idx_hbm, out_hbm, idx_v, f_v, e_v, acc_v, sem):
    c = lax.axis_index("c")
    s = lax.axis_index("s")
    wid = s * _NC + c
    base0 = wid * _PER_TILE
    iota = lax.iota(jnp.int32, _L)
    cols = [jnp.full((_L,), j, jnp.int32) for j in range(13)]

    def issue(b, base):
        pltpu.sync_copy(idx_hbm.at[pl.ds(base, _C)], idx_v.at[b])
        pltpu.async_copy(f_hbm.at[idx_v.at[b]], f_v.at[b], sem.at[b])
        for j in range(_NE):
            pltpu.async_copy(e_hbms[j].at[pl.ds(base, _C)],
                             e_v.at[b, j], sem.at[b])

    def drain(b):
        pltpu.make_async_copy(f_hbm.at[idx_v.at[b]], f_v.at[b],
                              sem.at[b]).wait()
        for j in range(_NE):
            pltpu.make_async_copy(e_hbms[j].at[pl.ds(0, _C)],
                                  e_v.at[b, j], sem.at[b]).wait()

    def compute(b, acc):
        def inner(i, acc):
            o = i * _L
            rows = iota + o
            r1 = e_v[b, 12, pl.ds(o, _L)]
            mm = e_v[b, 13, pl.ds(o, _L)]
            r2 = r1 * r1
            b0 = 1.0 / r1 + ((_ARFN * r2 + _ARFM) * r2 + _KRF) * r2 - _CRF
            rinv2 = 1.0 / r2
            b1 = b0 * rinv2
            b2 = 3.0 * b1 * rinv2
            fb = f_v.at[b]
            mono = plsc.load_gather(fb, [rows, cols[0]])
            dsum = (e_v[b, 0, pl.ds(o, _L)] *
                    plsc.load_gather(fb, [rows, cols[1]]))
            for j in range(1, 3):
                dsum = dsum + (e_v[b, j, pl.ds(o, _L)] *
                               plsc.load_gather(fb, [rows, cols[1 + j]]))
            qsum = (e_v[b, 3, pl.ds(o, _L)] *
                    plsc.load_gather(fb, [rows, cols[4]]))
            for j in range(1, 9):
                qsum = qsum + (e_v[b, 3 + j, pl.ds(o, _L)] *
                               plsc.load_gather(fb, [rows, cols[4 + j]]))
            return acc + mm * (mono * b0 + dsum * b1 + qsum * b2)

        return lax.fori_loop(0, _C // _L, inner, acc)

    issue(0, base0)

    def pair(k2, acc):
        base = base0 + (2 * k2) * _C
        issue(1, base + _C)
        drain(0)
        acc = compute(0, acc)

        @pl.when(2 * k2 + 2 < _NCHUNK)
        def _():
            issue(0, base + 2 * _C)

        drain(1)
        return compute(1, acc)

    acc = lax.fori_loop(0, _NCHUNK // 2, pair, jnp.zeros((_L,), jnp.float32))
    if _NCHUNK % 2:
        drain(0)
        acc = compute(0, acc)
    acc_v[...] = acc * _K_EPS
    pltpu.sync_copy(acc_v, out_hbm.at[pl.ds(wid * _L, _L)])


@functools.partial(
    pl.kernel,
    out_type=jax.ShapeDtypeStruct((_NW * _L,), jnp.float32),
    mesh=plsc.VectorSubcoreMesh(core_axis_name="c", subcore_axis_name="s",
                                num_cores=_NC, num_subcores=_NS),
    compiler_params=pltpu.CompilerParams(needs_layout_passes=False,
                                         use_tc_tiling_on_sc=False),
    scratch_types=[
        pltpu.VMEM((2, _C), jnp.int32),
        pltpu.VMEM((2, _C, _FW), jnp.float32),
        pltpu.VMEM((2, _NE, _C), jnp.float32),
        pltpu.VMEM((_L,), jnp.float32),
        pltpu.SemaphoreType.DMA((2,)),
    ],
)
def _sc_coulomb(f_hbm, e0, e1, e2, e3, e4, e5, e6, e7, e8, e9, e10, e11,
                e12, e13, idx_hbm, out_hbm, idx_v, f_v, e_v, acc_v, sem):
    _sc_body(f_hbm, (e0, e1, e2, e3, e4, e5, e6, e7, e8, e9, e10, e11,
                     e12, e13), idx_hbm, out_hbm, idx_v, f_v, e_v, acc_v, sem)


def kernel(monos, dipos, quads, Rx1_qmmm_esp, Rx2_qmmm_esp, R1_qmmm_esp,
           mm_monos_esp, receivers_qmmm_esp):
    n = monos.shape[0]
    feat = jnp.concatenate(
        [monos, dipos, quads.reshape(n, 9),
         jnp.zeros((n, _FW - 13), jnp.float32)], axis=1)
    idx_all = receivers_qmmm_esp.astype(jnp.int32)
    total = jnp.zeros((), jnp.float32)
    for p in range(_P):
        sl = slice(p * _EP, (p + 1) * _EP)
        comps = ([Rx1_qmmm_esp[sl, j] for j in range(3)] +
                 [Rx2_qmmm_esp[sl, i, j] for i in range(3) for j in range(3)] +
                 [R1_qmmm_esp[sl, 0], mm_monos_esp[sl, 0]])
        partials = _sc_coulomb(feat, *comps, idx_all[sl])
        total = total + jnp.sum(partials)
    return total.reshape(1, 1)


# speedup vs baseline: 1.7896x; 1.7896x over previous
import functools

import jax
import jax.numpy as jnp
from jax import lax
from jax.experimental import pallas as pl
from jax.experimental.pallas import tpu as pltpu
from jax.experimental.pallas import tpu_sc as plsc

_CUTOFF = 14.0
_EPS_RF = 78.4
_MRF = 4
_NRF = 6
_KRF = (_EPS_RF - 1) / (1 + 2 * _EPS_RF) * (1 / _CUTOFF ** 3)
_ARFM = 3 * _CUTOFF ** (-(_MRF + 1)) / (_MRF * (_NRF - _MRF)) * ((2 * _EPS_RF + _NRF - 1) / (1 + 2 * _EPS_RF))
_ARFN = 3 * _CUTOFF ** (-(_NRF + 1)) / (_NRF * (_MRF - _NRF)) * ((2 * _EPS_RF + _MRF - 1) / (1 + 2 * _EPS_RF))
_CRF = 3 * _EPS_RF / (1 + 2 * _EPS_RF) * (1 / _CUTOFF) + _ARFM * _CUTOFF ** _MRF + _ARFN * _CUTOFF ** _NRF
_K_EPS = 1389.35457644382

_NC = 2
_NS = 16
_NW = _NC * _NS
_L = 16

_N_EDGES = 3200000
_PER_TILE = _N_EDGES // _NW
_C = 2000
_NCHUNK = _PER_TILE // _C
_FW = 16
_NE = 14


def _sc_body(f_hbm, e_hbms, idx_hbm, out_hbm, idx_v, f_v, e_v, acc_v, sem):
    c = lax.axis_index("c")
    s = lax.axis_index("s")
    wid = s * _NC + c
    base0 = wid * _PER_TILE
    iota = lax.iota(jnp.int32, _L)
    cols = [jnp.full((_L,), j, jnp.int32) for j in range(13)]

    def issue(b, base):
        pltpu.sync_copy(idx_hbm.at[pl.ds(base, _C)], idx_v.at[b])
        pltpu.async_copy(f_hbm.at[idx_v.at[b]], f_v.at[b], sem.at[b])
        for j in range(_NE):
            pltpu.async_copy(e_hbms[j].at[pl.ds(base, _C)],
                             e_v.at[b, j], sem.at[b])

    def drain(b):
        pltpu.make_async_copy(f_hbm.at[idx_v.at[b]], f_v.at[b],
                              sem.at[b]).wait()
        for j in range(_NE):
            pltpu.make_async_copy(e_hbms[j].at[pl.ds(0, _C)],
                                  e_v.at[b, j], sem.at[b]).wait()

    def compute(b, acc):
        def inner(i, acc):
            o = i * _L
            rows = iota + o
            r1 = e_v[b, 12, pl.ds(o, _L)]
            mm = e_v[b, 13, pl.ds(o, _L)]
            r2 = r1 * r1
            b0 = 1.0 / r1 + ((_ARFN * r2 + _ARFM) * r2 + _KRF) * r2 - _CRF
            rinv2 = 1.0 / r2
            b1 = b0 * rinv2
            b2 = 3.0 * b1 * rinv2
            fb = f_v.at[b]
            mono = plsc.load_gather(fb, [rows, cols[0]])
            dsum = (e_v[b, 0, pl.ds(o, _L)] *
                    plsc.load_gather(fb, [rows, cols[1]]))
            for j in range(1, 3):
                dsum = dsum + (e_v[b, j, pl.ds(o, _L)] *
                               plsc.load_gather(fb, [rows, cols[1 + j]]))
            qsum = (e_v[b, 3, pl.ds(o, _L)] *
                    plsc.load_gather(fb, [rows, cols[4]]))
            for j in range(1, 9):
                qsum = qsum + (e_v[b, 3 + j, pl.ds(o, _L)] *
                               plsc.load_gather(fb, [rows, cols[4 + j]]))
            return acc + mm * (mono * b0 + dsum * b1 + qsum * b2)

        return lax.fori_loop(0, _C // _L, inner, acc)

    issue(0, base0)

    def pair(k2, acc):
        base = base0 + (2 * k2) * _C
        issue(1, base + _C)
        drain(0)
        acc = compute(0, acc)

        @pl.when(2 * k2 + 2 < _NCHUNK)
        def _():
            issue(0, base + 2 * _C)

        drain(1)
        return compute(1, acc)

    acc = lax.fori_loop(0, _NCHUNK // 2, pair, jnp.zeros((_L,), jnp.float32))
    if _NCHUNK % 2:
        drain(0)
        acc = compute(0, acc)
    acc_v[...] = acc * _K_EPS
    pltpu.sync_copy(acc_v, out_hbm.at[pl.ds(wid * _L, _L)])


@functools.partial(
    pl.kernel,
    out_type=jax.ShapeDtypeStruct((_NW * _L,), jnp.float32),
    mesh=plsc.VectorSubcoreMesh(core_axis_name="c", subcore_axis_name="s",
                                num_cores=_NC, num_subcores=_NS),
    compiler_params=pltpu.CompilerParams(needs_layout_passes=False,
                                         use_tc_tiling_on_sc=False),
    scratch_types=[
        pltpu.VMEM((2, _C), jnp.int32),
        pltpu.VMEM((2, _C, _FW), jnp.float32),
        pltpu.VMEM((2, _NE, _C), jnp.float32),
        pltpu.VMEM((_L,), jnp.float32),
        pltpu.SemaphoreType.DMA((2,)),
    ],
)
def _sc_coulomb(f_hbm, e0, e1, e2, e3, e4, e5, e6, e7, e8, e9, e10, e11,
                e12, e13, idx_hbm, out_hbm, idx_v, f_v, e_v, acc_v, sem):
    _sc_body(f_hbm, (e0, e1, e2, e3, e4, e5, e6, e7, e8, e9, e10, e11,
                     e12, e13), idx_hbm, out_hbm, idx_v, f_v, e_v, acc_v, sem)


def kernel(monos, dipos, quads, Rx1_qmmm_esp, Rx2_qmmm_esp, R1_qmmm_esp,
           mm_monos_esp, receivers_qmmm_esp):
    n = monos.shape[0]
    feat = jnp.concatenate(
        [monos, dipos, quads.reshape(n, 9),
         jnp.zeros((n, _FW - 13), jnp.float32)], axis=1)
    comps = ([Rx1_qmmm_esp[:, j] for j in range(3)] +
             [Rx2_qmmm_esp[:, i, j] for i in range(3) for j in range(3)] +
             [R1_qmmm_esp[:, 0], mm_monos_esp[:, 0]])
    partials = _sc_coulomb(feat, *comps,
                           receivers_qmmm_esp.astype(jnp.int32))
    return jnp.sum(partials).reshape(1, 1)


# speedup vs baseline: 1.7904x; 1.0004x over previous
import functools

import jax
import jax.numpy as jnp
from jax import lax
from jax.experimental import pallas as pl
from jax.experimental.pallas import tpu as pltpu
from jax.experimental.pallas import tpu_sc as plsc

_CUTOFF = 14.0
_EPS_RF = 78.4
_MRF = 4
_NRF = 6
_KRF = (_EPS_RF - 1) / (1 + 2 * _EPS_RF) * (1 / _CUTOFF ** 3)
_ARFM = 3 * _CUTOFF ** (-(_MRF + 1)) / (_MRF * (_NRF - _MRF)) * ((2 * _EPS_RF + _NRF - 1) / (1 + 2 * _EPS_RF))
_ARFN = 3 * _CUTOFF ** (-(_NRF + 1)) / (_NRF * (_MRF - _NRF)) * ((2 * _EPS_RF + _MRF - 1) / (1 + 2 * _EPS_RF))
_CRF = 3 * _EPS_RF / (1 + 2 * _EPS_RF) * (1 / _CUTOFF) + _ARFM * _CUTOFF ** _MRF + _ARFN * _CUTOFF ** _NRF
_K_EPS = 1389.35457644382

_NC = 2
_NS = 16
_NW = _NC * _NS
_L = 16

_N_EDGES = 3200000
_PER_TILE = _N_EDGES // _NW
_C = 2000
_NCHUNK = _PER_TILE // _C
_FW = 16
_NE = 14


def _sc_body(f_hbm, e_hbms, idx_hbm, out_hbm, idx_v, f_v, e_v, acc_v, sem):
    c = lax.axis_index("c")
    s = lax.axis_index("s")
    wid = s * _NC + c
    base0 = wid * _PER_TILE
    iota = lax.iota(jnp.int32, _L)
    cols = [jnp.full((_L,), j, jnp.int32) for j in range(13)]

    def issue(b, base):
        pltpu.sync_copy(idx_hbm.at[pl.ds(base, _C)], idx_v.at[b])
        pltpu.async_copy(f_hbm.at[idx_v.at[b]], f_v.at[b], sem.at[b])
        for j in range(_NE):
            pltpu.async_copy(e_hbms[j].at[pl.ds(base, _C)],
                             e_v.at[b, j], sem.at[b])

    def drain(b):
        pltpu.make_async_copy(f_hbm.at[idx_v.at[b]], f_v.at[b],
                              sem.at[b]).wait()
        for j in range(_NE):
            pltpu.make_async_copy(e_hbms[j].at[pl.ds(0, _C)],
                                  e_v.at[b, j], sem.at[b]).wait()

    def compute(b, acc):
        fb = f_v.at[b]

        def inner(i, acc):
            o = i * _L
            rows = iota + o
            r1 = e_v[b, 12, pl.ds(o, _L)]
            mm = e_v[b, 13, pl.ds(o, _L)]
            r2 = r1 * r1
            b0 = 1.0 / r1 + ((_ARFN * r2 + _ARFM) * r2 + _KRF) * r2 - _CRF
            rinv2 = 1.0 / r2
            b1 = b0 * rinv2
            b2 = 3.0 * b1 * rinv2
            mono = plsc.load_gather(fb, [rows, cols[0]])
            dsum = (e_v[b, 0, pl.ds(o, _L)] *
                    plsc.load_gather(fb, [rows, cols[1]]))
            for j in range(1, 3):
                dsum = dsum + (e_v[b, j, pl.ds(o, _L)] *
                               plsc.load_gather(fb, [rows, cols[1 + j]]))
            qsum = (e_v[b, 3, pl.ds(o, _L)] *
                    plsc.load_gather(fb, [rows, cols[4]]))
            for j in range(1, 9):
                qsum = qsum + (e_v[b, 3 + j, pl.ds(o, _L)] *
                               plsc.load_gather(fb, [rows, cols[4 + j]]))
            return acc + mm * (mono * b0 + dsum * b1 + qsum * b2)

        return lax.fori_loop(0, _C // _L, inner, acc)

    issue(0, base0)

    def pair(k2, acc):
        base = base0 + (2 * k2) * _C
        issue(1, base + _C)
        drain(0)
        acc = compute(0, acc)

        @pl.when(2 * k2 + 2 < _NCHUNK)
        def _():
            issue(0, base + 2 * _C)

        drain(1)
        return compute(1, acc)

    acc = lax.fori_loop(0, _NCHUNK // 2, pair, jnp.zeros((_L,), jnp.float32))
    if _NCHUNK % 2:
        drain(0)
        acc = compute(0, acc)
    acc_v[...] = acc * _K_EPS
    pltpu.sync_copy(acc_v, out_hbm.at[pl.ds(wid * _L, _L)])


@functools.partial(
    pl.kernel,
    out_type=jax.ShapeDtypeStruct((_NW * _L,), jnp.float32),
    mesh=plsc.VectorSubcoreMesh(core_axis_name="c", subcore_axis_name="s",
                                num_cores=_NC, num_subcores=_NS),
    compiler_params=pltpu.CompilerParams(needs_layout_passes=False,
                                         use_tc_tiling_on_sc=False),
    scratch_types=[
        pltpu.VMEM((2, _C), jnp.int32),
        pltpu.VMEM((2, _C, _FW), jnp.float32),
        pltpu.VMEM((2, _NE, _C), jnp.float32),
        pltpu.VMEM((_L,), jnp.float32),
        pltpu.SemaphoreType.DMA((2,)),
    ],
)
def _sc_coulomb(f_hbm, e0, e1, e2, e3, e4, e5, e6, e7, e8, e9, e10, e11,
                e12, e13, idx_hbm, out_hbm, idx_v, f_v, e_v, acc_v, sem):
    _sc_body(f_hbm, (e0, e1, e2, e3, e4, e5, e6, e7, e8, e9, e10, e11,
                     e12, e13), idx_hbm, out_hbm, idx_v, f_v, e_v, acc_v, sem)


def kernel(monos, dipos, quads, Rx1_qmmm_esp, Rx2_qmmm_esp, R1_qmmm_esp,
           mm_monos_esp, receivers_qmmm_esp):
    n = monos.shape[0]
    feat = jnp.concatenate(
        [monos, dipos, quads.reshape(n, 9),
         jnp.zeros((n, _FW - 13), jnp.float32)], axis=1)
    comps = ([Rx1_qmmm_esp[:, j] for j in range(3)] +
             [Rx2_qmmm_esp[:, i, j] for i in range(3) for j in range(3)] +
             [R1_qmmm_esp[:, 0], mm_monos_esp[:, 0]])
    partials = _sc_coulomb(feat, *comps,
                           receivers_qmmm_esp.astype(jnp.int32))
    return jnp.sum(partials).reshape(1, 1)


# speedup vs baseline: 1.9044x; 1.0637x over previous
import functools

import jax
import jax.numpy as jnp
from jax import lax
from jax.experimental import pallas as pl
from jax.experimental.pallas import tpu as pltpu
from jax.experimental.pallas import tpu_sc as plsc

_CUTOFF = 14.0
_EPS_RF = 78.4
_MRF = 4
_NRF = 6
_KRF = (_EPS_RF - 1) / (1 + 2 * _EPS_RF) * (1 / _CUTOFF ** 3)
_ARFM = 3 * _CUTOFF ** (-(_MRF + 1)) / (_MRF * (_NRF - _MRF)) * ((2 * _EPS_RF + _NRF - 1) / (1 + 2 * _EPS_RF))
_ARFN = 3 * _CUTOFF ** (-(_NRF + 1)) / (_NRF * (_MRF - _NRF)) * ((2 * _EPS_RF + _MRF - 1) / (1 + 2 * _EPS_RF))
_CRF = 3 * _EPS_RF / (1 + 2 * _EPS_RF) * (1 / _CUTOFF) + _ARFM * _CUTOFF ** _MRF + _ARFN * _CUTOFF ** _NRF
_K_EPS = 1389.35457644382

_NC = 2
_NS = 16
_NW = _NC * _NS
_L = 16

_N_EDGES = 3200000
_PER_TILE = _N_EDGES // _NW
_C = 2000
_NCHUNK = _PER_TILE // _C
_FW = 16
_NE = 14


def _sc_body(f_hbm, e_hbms, idx_hbm, out_hbm, idx_v, f_v, e_v, acc_v, sem,
             isem):
    c = lax.axis_index("c")
    s = lax.axis_index("s")
    wid = s * _NC + c
    base0 = wid * _PER_TILE
    iota = lax.iota(jnp.int32, _L)
    cols = [jnp.full((_L,), j, jnp.int32) for j in range(13)]

    def idx_issue(b, base):
        pltpu.async_copy(idx_hbm.at[pl.ds(base, _C)], idx_v.at[b],
                         isem.at[b])

    def idx_drain(b):
        pltpu.make_async_copy(idx_hbm.at[pl.ds(0, _C)], idx_v.at[b],
                              isem.at[b]).wait()

    def issue(b, base):
        pltpu.async_copy(f_hbm.at[idx_v.at[b]], f_v.at[b], sem.at[b])
        for j in range(_NE):
            pltpu.async_copy(e_hbms[j].at[pl.ds(base, _C)],
                             e_v.at[b, j], sem.at[b])

    def drain(b):
        pltpu.make_async_copy(f_hbm.at[idx_v.at[b]], f_v.at[b],
                              sem.at[b]).wait()
        for j in range(_NE):
            pltpu.make_async_copy(e_hbms[j].at[pl.ds(0, _C)],
                                  e_v.at[b, j], sem.at[b]).wait()

    def compute(b, acc):
        fb = f_v.at[b]

        def inner(i, acc):
            o = i * _L
            rows = iota + o
            r1 = e_v[b, 12, pl.ds(o, _L)]
            mm = e_v[b, 13, pl.ds(o, _L)]
            r2 = r1 * r1
            b0 = 1.0 / r1 + ((_ARFN * r2 + _ARFM) * r2 + _KRF) * r2 - _CRF
            rinv2 = 1.0 / r2
            b1 = b0 * rinv2
            b2 = 3.0 * b1 * rinv2
            mono = plsc.load_gather(fb, [rows, cols[0]])
            dsum = (e_v[b, 0, pl.ds(o, _L)] *
                    plsc.load_gather(fb, [rows, cols[1]]))
            for j in range(1, 3):
                dsum = dsum + (e_v[b, j, pl.ds(o, _L)] *
                               plsc.load_gather(fb, [rows, cols[1 + j]]))
            qsum = (e_v[b, 3, pl.ds(o, _L)] *
                    plsc.load_gather(fb, [rows, cols[4]]))
            for j in range(1, 9):
                qsum = qsum + (e_v[b, 3 + j, pl.ds(o, _L)] *
                               plsc.load_gather(fb, [rows, cols[4 + j]]))
            return acc + mm * (mono * b0 + dsum * b1 + qsum * b2)

        return lax.fori_loop(0, _C // _L, inner, acc)

    assert _NCHUNK % 2 == 0
    idx_issue(0, base0)
    idx_drain(0)
    issue(0, base0)
    idx_issue(1, base0 + _C)

    def pair(k2, acc):
        base = base0 + (2 * k2) * _C
        idx_drain(1)
        issue(1, base + _C)
        drain(0)

        @pl.when(2 * k2 + 2 < _NCHUNK)
        def _():
            idx_issue(0, base + 2 * _C)

        acc = compute(0, acc)

        @pl.when(2 * k2 + 2 < _NCHUNK)
        def _():
            idx_drain(0)
            issue(0, base + 2 * _C)

        drain(1)

        @pl.when(2 * k2 + 3 < _NCHUNK)
        def _():
            idx_issue(1, base + 3 * _C)

        return compute(1, acc)

    acc = lax.fori_loop(0, _NCHUNK // 2, pair, jnp.zeros((_L,), jnp.float32))
    acc_v[...] = acc * _K_EPS
    pltpu.sync_copy(acc_v, out_hbm.at[pl.ds(wid * _L, _L)])


@functools.partial(
    pl.kernel,
    out_type=jax.ShapeDtypeStruct((_NW * _L,), jnp.float32),
    mesh=plsc.VectorSubcoreMesh(core_axis_name="c", subcore_axis_name="s",
                                num_cores=_NC, num_subcores=_NS),
    compiler_params=pltpu.CompilerParams(needs_layout_passes=False,
                                         use_tc_tiling_on_sc=False),
    scratch_types=[
        pltpu.VMEM((2, _C), jnp.int32),
        pltpu.VMEM((2, _C, _FW), jnp.float32),
        pltpu.VMEM((2, _NE, _C), jnp.float32),
        pltpu.VMEM((_L,), jnp.float32),
        pltpu.SemaphoreType.DMA((2,)),
        pltpu.SemaphoreType.DMA((2,)),
    ],
)
def _sc_coulomb(f_hbm, e0, e1, e2, e3, e4, e5, e6, e7, e8, e9, e10, e11,
                e12, e13, idx_hbm, out_hbm, idx_v, f_v, e_v, acc_v, sem,
                isem):
    _sc_body(f_hbm, (e0, e1, e2, e3, e4, e5, e6, e7, e8, e9, e10, e11,
                     e12, e13), idx_hbm, out_hbm, idx_v, f_v, e_v, acc_v,
             sem, isem)


def kernel(monos, dipos, quads, Rx1_qmmm_esp, Rx2_qmmm_esp, R1_qmmm_esp,
           mm_monos_esp, receivers_qmmm_esp):
    n = monos.shape[0]
    feat = jnp.concatenate(
        [monos, dipos, quads.reshape(n, 9),
         jnp.zeros((n, _FW - 13), jnp.float32)], axis=1)
    comps = ([Rx1_qmmm_esp[:, j] for j in range(3)] +
             [Rx2_qmmm_esp[:, i, j] for i in range(3) for j in range(3)] +
             [R1_qmmm_esp[:, 0], mm_monos_esp[:, 0]])
    partials = _sc_coulomb(feat, *comps,
                           receivers_qmmm_esp.astype(jnp.int32))
    return jnp.sum(partials).reshape(1, 1)
